# Initial kernel scaffold; baseline (speedup 1.0000x reference)
import jax
import jax.numpy as jnp
from jax.experimental import pallas as pl


def kernel(x, edge_index, W_lin, b_lin, W1l, b1, W1r, W2l, b2, W2r, Wc):
    raise NotImplementedError("write your pallas kernel here")


# speedup vs baseline: 3.8539x; 3.8539x over previous
import functools

import jax
import jax.numpy as jnp
from jax import lax
from jax.experimental import pallas as pl
from jax.experimental.pallas import tpu as pltpu
from jax.experimental.pallas import tpu_sc as plsc

D = 128
N_PAD = 10240
NC, NS = 2, 16
NW = NC * NS
CHUNK = 128
ROWS_PER_TILE = N_PAD // NS

_MESH = plsc.VectorSubcoreMesh(core_axis_name="c", subcore_axis_name="s")


def _make_sc_agg(n_chunk: int):
    def body(h_hbm, src_hbm, dst_hbm, zero_hbm, agg_out,
             src_v, dst_v, rows_v, agg_sh, sem):
        c = lax.axis_index("c")
        s = lax.axis_index("s")
        w = c * NS + s
        pltpu.sync_copy(src_hbm.at[w], src_v)
        pltpu.sync_copy(dst_hbm.at[w], dst_v)
        r0 = s * ROWS_PER_TILE
        pltpu.sync_copy(zero_hbm.at[pl.ds(r0, ROWS_PER_TILE)],
                        agg_sh.at[pl.ds(r0, ROWS_PER_TILE)])
        plsc.subcore_barrier()

        def step(i, carry):
            pltpu.async_copy(h_hbm.at[src_v.at[i]], rows_v, sem).wait()
            pltpu.sync_copy(rows_v, agg_sh.at[dst_v.at[i]], add=True)
            return carry

        lax.fori_loop(0, n_chunk, step, 0)
        plsc.subcore_barrier()
        pltpu.sync_copy(agg_sh.at[pl.ds(r0, ROWS_PER_TILE)],
                        agg_out.at[c, pl.ds(r0, ROWS_PER_TILE)])

    return pl.kernel(
        body,
        out_type=jax.ShapeDtypeStruct((NC, N_PAD, D), jnp.float32),
        mesh=_MESH,
        scratch_types=[
            pltpu.VMEM((n_chunk, CHUNK), jnp.int32),
            pltpu.VMEM((n_chunk, CHUNK), jnp.int32),
            pltpu.VMEM((CHUNK, D), jnp.float32),
            pltpu.VMEM_SHARED((N_PAD, D), jnp.float32),
            pltpu.SemaphoreType.DMA,
        ],
        name="sc_agg")


def _make_sc_deg(n_chunk: int):
    def body(dst_hbm, zero_hbm, ones_hbm, deg_out, dst_v, ones_v, deg_sh):
        c = lax.axis_index("c")
        s = lax.axis_index("s")
        w = c * NS + s
        pltpu.sync_copy(dst_hbm.at[w], dst_v)
        pltpu.sync_copy(ones_hbm, ones_v)
        r0 = s * ROWS_PER_TILE
        pltpu.sync_copy(zero_hbm.at[pl.ds(r0, ROWS_PER_TILE)],
                        deg_sh.at[pl.ds(r0, ROWS_PER_TILE)])
        plsc.subcore_barrier()

        def step(i, carry):
            pltpu.sync_copy(ones_v, deg_sh.at[dst_v.at[i]], add=True)
            return carry

        lax.fori_loop(0, n_chunk, step, 0)
        plsc.subcore_barrier()
        pltpu.sync_copy(deg_sh.at[pl.ds(r0, ROWS_PER_TILE)],
                        deg_out.at[c, pl.ds(r0, ROWS_PER_TILE)])

    return pl.kernel(
        body,
        out_type=jax.ShapeDtypeStruct((NC, N_PAD, D), jnp.float32),
        mesh=_MESH,
        scratch_types=[
            pltpu.VMEM((n_chunk, CHUNK), jnp.int32),
            pltpu.VMEM((CHUNK, D), jnp.float32),
            pltpu.VMEM_SHARED((N_PAD, D), jnp.float32),
        ],
        name="sc_deg")


BM = 1024
_GRID = (N_PAD // BM,)
_row = lambda k: pl.BlockSpec((BM, k), lambda i: (i, 0))
_full = lambda shape: pl.BlockSpec(shape, lambda i: (0,) * len(shape))


def _tc_in_body(x_ref, w_ref, b_ref, o_ref):
    o_ref[...] = jnp.maximum(
        jnp.dot(x_ref[...], w_ref[...], preferred_element_type=jnp.float32)
        + b_ref[...], 0.0)


def _tc_in(x, wt, b):
    return pl.pallas_call(
        _tc_in_body, grid=_GRID,
        in_specs=[_row(D), _full((D, D)), _full((1, D))],
        out_specs=_row(D),
        out_shape=jax.ShapeDtypeStruct((N_PAD, D), jnp.float32),
    )(x, wt, b)


def _tc_sage_body(final, a0, a1, d0, d1, h_ref, wl, b, wr, *rest):
    deg = d0[...][:, 0:1] + d1[...][:, 0:1]
    mean = (a0[...] + a1[...]) / jnp.maximum(deg, 1.0)
    t = (jnp.dot(mean, wl[...], preferred_element_type=jnp.float32) + b[...]
         + jnp.dot(h_ref[...], wr[...], preferred_element_type=jnp.float32))
    if final:
        wc, o_ref = rest
        o_ref[...] = jnp.dot(jnp.maximum(t, 0.0), wc[...],
                             preferred_element_type=jnp.float32)
    else:
        (o_ref,) = rest
        o_ref[...] = t


def _tc_sage_mid(a0, a1, d0, d1, h, wlt, b, wrt):
    return pl.pallas_call(
        functools.partial(_tc_sage_body, False), grid=_GRID,
        in_specs=[_row(D), _row(D), _row(D), _row(D), _row(D),
                  _full((D, D)), _full((1, D)), _full((D, D))],
        out_specs=_row(D),
        out_shape=jax.ShapeDtypeStruct((N_PAD, D), jnp.float32),
    )(a0, a1, d0, d1, h, wlt, b, wrt)


def _tc_sage_out(a0, a1, d0, d1, h, wlt, b, wrt, wct, nclass):
    return pl.pallas_call(
        functools.partial(_tc_sage_body, True), grid=_GRID,
        in_specs=[_row(D), _row(D), _row(D), _row(D), _row(D),
                  _full((D, D)), _full((1, D)), _full((D, D)),
                  _full((D, nclass))],
        out_specs=_row(nclass),
        out_shape=jax.ShapeDtypeStruct((N_PAD, nclass), jnp.float32),
    )(a0, a1, d0, d1, h, wlt, b, wrt, wct)


def kernel(x, edge_index, W_lin, b_lin, W1l, b1, W1r, W2l, b2, W2r, Wc):
    n = x.shape[0]
    e = edge_index.shape[1]
    nclass = Wc.shape[0]
    n_chunk = -(-e // (NW * CHUNK))
    e_pad = NW * CHUNK * n_chunk
    sc_agg = _make_sc_agg(n_chunk)
    sc_deg = _make_sc_deg(n_chunk)

    src = jnp.concatenate(
        [edge_index[0], jnp.zeros((e_pad - e,), jnp.int32)]
    ).reshape(NW, n_chunk, CHUNK)
    dst = jnp.concatenate(
        [edge_index[1], jnp.full((e_pad - e,), N_PAD - 1, jnp.int32)]
    ).reshape(NW, n_chunk, CHUNK)
    xp = jnp.pad(x, ((0, N_PAD - n), (0, 0)))
    zero = jnp.zeros((N_PAD, D), jnp.float32)
    ones = jnp.ones((CHUNK, D), jnp.float32)

    h0 = _tc_in(xp, W_lin.T, b_lin.reshape(1, D))
    deg = sc_deg(dst, zero, ones)
    agg1 = sc_agg(h0, src, dst, zero)
    h1 = _tc_sage_mid(agg1[0], agg1[1], deg[0], deg[1], h0,
                      W1l.T, b1.reshape(1, D), W1r.T)
    agg2 = sc_agg(h1, src, dst, zero)
    out = _tc_sage_out(agg2[0], agg2[1], deg[0], deg[1], h1,
                       W2l.T, b2.reshape(1, D), W2r.T, Wc.T, nclass)
    return out[:n]
